# Initial kernel scaffold; baseline (speedup 1.0000x reference)
import jax
import jax.numpy as jnp
from jax.experimental import pallas as pl


def kernel(x, idx, Wq, bq, Wk, bk, Wv, bv, W1, b1, W2, b2, g1, be1, g2, be2):
    raise NotImplementedError("write your pallas kernel here")


# speedup vs baseline: 403.9439x; 403.9439x over previous
import functools
import math

import jax
import jax.numpy as jnp
from jax import lax
from jax.experimental import pallas as pl
from jax.experimental.pallas import tpu as pltpu
from jax.experimental.pallas import tpu_sc as plsc

L = 2048
D = 768
U = 15615
_TOTAL = L * U
_ROWBUF = 15648
_NW = 32
_RPW = L // _NW


def _sc_histogram(idx_flat):
    mesh = plsc.VectorSubcoreMesh(
        core_axis_name="c", subcore_axis_name="s", num_cores=2, num_subcores=16
    )

    @functools.partial(
        pl.kernel,
        out_type=jax.ShapeDtypeStruct((L, L), jnp.float32),
        mesh=mesh,
        compiler_params=pltpu.CompilerParams(needs_layout_passes=False),
        scratch_types=[
            pltpu.VMEM((_ROWBUF,), jnp.int32),
            pltpu.VMEM((_ROWBUF,), jnp.int32),
            pltpu.VMEM((L,), jnp.float32),
            pltpu.SemaphoreType.DMA,
            pltpu.SemaphoreType.DMA,
        ],
    )
    def hist(idx_hbm, out_hbm, buf0, buf1, counts, sem0, sem1):
        wid = lax.axis_index("s") * 2 + lax.axis_index("c")
        base = wid * _RPW
        ones16 = jnp.ones((16,), jnp.float32)
        zeros16 = jnp.zeros((16,), jnp.float32)
        tail_mask = jnp.arange(16, dtype=jnp.int32) < (U - (U // 16) * 16)

        def row_start(r):
            rs = r * U
            st = pl.multiple_of(jnp.minimum(rs & -16, _TOTAL - _ROWBUF), 16)
            return st, rs - st

        def start_copy(r, buf, sem):
            st, _ = row_start(r)
            pltpu.async_copy(idx_hbm.at[pl.ds(st, _ROWBUF)], buf, sem)

        def zbody(i, _):
            counts[pl.ds(i * 16, 16)] = zeros16
            return 0

        lax.fori_loop(0, L // 16, zbody, 0)
        start_copy(base, buf0, sem0)
        start_copy(base + 1, buf1, sem1)

        def process(r, buf, sem):
            pltpu.make_async_copy(idx_hbm.at[pl.ds(0, _ROWBUF)], buf, sem).wait()
            _, off = row_start(r)

            def sbody(j, _):
                iv = buf[pl.ds(off + j * 16, 16)]
                plsc.addupdate_scatter(counts, [iv], ones16)
                return 0

            lax.fori_loop(0, U // 16, sbody, 0, unroll=8)
            iv = buf[pl.ds(off + (U // 16) * 16, 16)]
            plsc.addupdate_scatter(counts, [iv], ones16, mask=tail_mask)
            pltpu.sync_copy(counts, out_hbm.at[r])
            lax.fori_loop(0, L // 16, zbody, 0)

        def body(i, _):
            r0 = base + 2 * i
            process(r0, buf0, sem0)
            start_copy(r0 + 2, buf0, sem0)
            process(r0 + 1, buf1, sem1)
            start_copy(r0 + 3, buf1, sem1)
            return 0

        lax.fori_loop(0, _RPW // 2, body, 0)
        pltpu.make_async_copy(idx_hbm.at[pl.ds(0, _ROWBUF)], buf0, sem0).wait()
        pltpu.make_async_copy(idx_hbm.at[pl.ds(0, _ROWBUF)], buf1, sem1).wait()

    return hist(idx_flat)


def _tc_qkv(x, wqkv, bqkv, tm=256):
    def body(x_ref, w_ref, b_ref, q_ref, k_ref, v_ref):
        y = (
            jnp.dot(x_ref[...], w_ref[...], preferred_element_type=jnp.float32)
            + b_ref[...]
        )
        q_ref[...] = y[:, :D]
        k_ref[...] = y[:, D : 2 * D]
        v_ref[...] = y[:, 2 * D :]

    out = pl.pallas_call(
        body,
        grid=(L // tm,),
        in_specs=[
            pl.BlockSpec((tm, D), lambda i: (i, 0)),
            pl.BlockSpec((D, 3 * D), lambda i: (0, 0)),
            pl.BlockSpec((1, 3 * D), lambda i: (0, 0)),
        ],
        out_specs=[
            pl.BlockSpec((tm, D), lambda i: (i, 0)),
            pl.BlockSpec((tm, D), lambda i: (i, 0)),
            pl.BlockSpec((tm, D), lambda i: (i, 0)),
        ],
        out_shape=[jax.ShapeDtypeStruct((L, D), jnp.float32)] * 3,
    )(x, wqkv, bqkv)
    return out


def _tc_attn(q, k, v, cnt, x, g1, be1, tm=256):
    inv_sqrt_d = 1.0 / math.sqrt(D)

    def body(q_ref, k_ref, v_ref, c_ref, x_ref, g_ref, b_ref, o_ref):
        s = (
            lax.dot_general(
                q_ref[...],
                k_ref[...],
                (((1,), (1,)), ((), ())),
                preferred_element_type=jnp.float32,
            )
            * inv_sqrt_d
        )
        c = c_ref[...]
        covered = c > 0.0
        cf = covered.astype(jnp.float32)
        mx = jnp.max(jnp.where(covered, s, -1e30), axis=1)
        ws = jnp.sum(c * s, axis=1) * (1.0 / U)
        nc = jnp.sum(cf, axis=1)
        vv = v_ref[...]
        vcov = jnp.dot(cf, vv, preferred_element_type=jnp.float32)
        vtot = jnp.sum(vv, axis=0, keepdims=True)
        em = jnp.exp(ws - mx)
        denom = nc + em * (L - nc)
        z = (vcov + em[:, None] * (vtot - vcov)) / denom[:, None]
        y = x_ref[...] + z
        mu = jnp.mean(y, axis=1, keepdims=True)
        var = jnp.mean((y - mu) ** 2, axis=1, keepdims=True)
        o_ref[...] = (y - mu) * lax.rsqrt(var + 1e-5) * g_ref[...] + b_ref[...]

    return pl.pallas_call(
        body,
        grid=(L // tm,),
        in_specs=[
            pl.BlockSpec((tm, D), lambda i: (i, 0)),
            pl.BlockSpec((L, D), lambda i: (0, 0)),
            pl.BlockSpec((L, D), lambda i: (0, 0)),
            pl.BlockSpec((tm, L), lambda i: (i, 0)),
            pl.BlockSpec((tm, D), lambda i: (i, 0)),
            pl.BlockSpec((1, D), lambda i: (0, 0)),
            pl.BlockSpec((1, D), lambda i: (0, 0)),
        ],
        out_specs=pl.BlockSpec((tm, D), lambda i: (i, 0)),
        out_shape=jax.ShapeDtypeStruct((L, D), jnp.float32),
    )(q, k, v, cnt, x, g1, be1)


def _tc_ffn(x1, w1, b1, w2, b2, g2, be2, tm=256):
    inv_sqrt2 = 1.0 / math.sqrt(2.0)

    def body(x_ref, w1_ref, b1_ref, w2_ref, b2_ref, g_ref, be_ref, o_ref):
        xx = x_ref[...]
        h = (
            jnp.dot(xx, w1_ref[...], preferred_element_type=jnp.float32)
            + b1_ref[...]
        )
        h = 0.5 * h * (1.0 + lax.erf(h * inv_sqrt2))
        y = (
            jnp.dot(h, w2_ref[...], preferred_element_type=jnp.float32)
            + b2_ref[...]
            + xx
        )
        mu = jnp.mean(y, axis=1, keepdims=True)
        var = jnp.mean((y - mu) ** 2, axis=1, keepdims=True)
        o_ref[...] = (y - mu) * lax.rsqrt(var + 1e-5) * g_ref[...] + be_ref[...]

    return pl.pallas_call(
        body,
        grid=(L // tm,),
        in_specs=[
            pl.BlockSpec((tm, D), lambda i: (i, 0)),
            pl.BlockSpec((D, 4 * D), lambda i: (0, 0)),
            pl.BlockSpec((1, 4 * D), lambda i: (0, 0)),
            pl.BlockSpec((4 * D, D), lambda i: (0, 0)),
            pl.BlockSpec((1, D), lambda i: (0, 0)),
            pl.BlockSpec((1, D), lambda i: (0, 0)),
            pl.BlockSpec((1, D), lambda i: (0, 0)),
        ],
        out_specs=pl.BlockSpec((tm, D), lambda i: (i, 0)),
        out_shape=jax.ShapeDtypeStruct((L, D), jnp.float32),
    )(x1, w1, b1, w2, b2, g2, be2)


def kernel(x, idx, Wq, bq, Wk, bk, Wv, bv, W1, b1, W2, b2, g1, be1, g2, be2):
    x2d = x[0]
    cnt = _sc_histogram(idx.reshape(-1))
    wqkv = jnp.concatenate([Wq, Wk, Wv], axis=1)
    bqkv = jnp.concatenate([bq, bk, bv])[None, :]
    q, k, v = _tc_qkv(x2d, wqkv, bqkv)
    x1 = _tc_attn(q, k, v, cnt, x2d, g1[None], be1[None])
    x2 = _tc_ffn(x1, W1, b1[None], W2, b2[None], g2[None], be2[None])
    return x2[None]


# speedup vs baseline: 478.5774x; 1.1848x over previous
import functools
import math

import jax
import jax.numpy as jnp
from jax import lax
from jax.experimental import pallas as pl
from jax.experimental.pallas import tpu as pltpu
from jax.experimental.pallas import tpu_sc as plsc

L = 2048
D = 768
U = 15615
_TOTAL = L * U
_ROWBUF = 15648
_NW = 32
_RPW = L // _NW


def _sc_histogram(idx_flat):
    mesh = plsc.VectorSubcoreMesh(
        core_axis_name="c", subcore_axis_name="s", num_cores=2, num_subcores=16
    )

    @functools.partial(
        pl.kernel,
        out_type=jax.ShapeDtypeStruct((L, L), jnp.float32),
        mesh=mesh,
        compiler_params=pltpu.CompilerParams(needs_layout_passes=False),
        scratch_types=[
            pltpu.VMEM((_ROWBUF,), jnp.int32),
            pltpu.VMEM((_ROWBUF,), jnp.int32),
            pltpu.VMEM((L,), jnp.float32),
            pltpu.SemaphoreType.DMA,
            pltpu.SemaphoreType.DMA,
        ],
    )
    def hist(idx_hbm, out_hbm, buf0, buf1, counts, sem0, sem1):
        wid = lax.axis_index("s") * 2 + lax.axis_index("c")
        base = wid * _RPW
        ones16 = jnp.ones((16,), jnp.float32)
        zeros16 = jnp.zeros((16,), jnp.float32)
        tail_mask = jnp.arange(16, dtype=jnp.int32) < (U - (U // 16) * 16)

        def row_start(r):
            rs = r * U
            st = pl.multiple_of(jnp.minimum(rs & -16, _TOTAL - _ROWBUF), 16)
            return st, rs - st

        def start_copy(r, buf, sem):
            st, _ = row_start(r)
            pltpu.async_copy(idx_hbm.at[pl.ds(st, _ROWBUF)], buf, sem)

        def zero_counts():
            @plsc.parallel_loop(0, L // 16, unroll=8)
            def _(i):
                counts[pl.ds(i * 16, 16)] = zeros16

        zero_counts()
        start_copy(base, buf0, sem0)
        start_copy(base + 1, buf1, sem1)

        def process(r, buf, sem):
            pltpu.make_async_copy(idx_hbm.at[pl.ds(0, _ROWBUF)], buf, sem).wait()
            _, off = row_start(r)

            @plsc.parallel_loop(0, U // 16, unroll=8)
            def _(j):
                iv = buf[pl.ds(off + j * 16, 16)]
                plsc.addupdate_scatter(counts, [iv], ones16)

            iv = buf[pl.ds(off + (U // 16) * 16, 16)]
            plsc.addupdate_scatter(counts, [iv], ones16, mask=tail_mask)
            pltpu.sync_copy(counts, out_hbm.at[r])
            zero_counts()

        def body(i, _):
            r0 = base + 2 * i
            process(r0, buf0, sem0)
            start_copy(r0 + 2, buf0, sem0)
            process(r0 + 1, buf1, sem1)
            start_copy(r0 + 3, buf1, sem1)
            return 0

        lax.fori_loop(0, _RPW // 2, body, 0)
        pltpu.make_async_copy(idx_hbm.at[pl.ds(0, _ROWBUF)], buf0, sem0).wait()
        pltpu.make_async_copy(idx_hbm.at[pl.ds(0, _ROWBUF)], buf1, sem1).wait()

    return hist(idx_flat)


def _tc_qkv(x, wqkv, bqkv, tm=256):
    def body(x_ref, w_ref, b_ref, q_ref, k_ref, v_ref):
        y = (
            jnp.dot(x_ref[...], w_ref[...], preferred_element_type=jnp.float32)
            + b_ref[...]
        )
        q_ref[...] = y[:, :D]
        k_ref[...] = y[:, D : 2 * D]
        v_ref[...] = y[:, 2 * D :]

    out = pl.pallas_call(
        body,
        grid=(L // tm,),
        in_specs=[
            pl.BlockSpec((tm, D), lambda i: (i, 0)),
            pl.BlockSpec((D, 3 * D), lambda i: (0, 0)),
            pl.BlockSpec((1, 3 * D), lambda i: (0, 0)),
        ],
        out_specs=[
            pl.BlockSpec((tm, D), lambda i: (i, 0)),
            pl.BlockSpec((tm, D), lambda i: (i, 0)),
            pl.BlockSpec((tm, D), lambda i: (i, 0)),
        ],
        out_shape=[jax.ShapeDtypeStruct((L, D), jnp.float32)] * 3,
    )(x, wqkv, bqkv)
    return out


def _tc_attn(q, k, v, cnt, x, g1, be1, tm=256):
    inv_sqrt_d = 1.0 / math.sqrt(D)

    def body(q_ref, k_ref, v_ref, c_ref, x_ref, g_ref, b_ref, o_ref):
        s = (
            lax.dot_general(
                q_ref[...],
                k_ref[...],
                (((1,), (1,)), ((), ())),
                preferred_element_type=jnp.float32,
            )
            * inv_sqrt_d
        )
        c = c_ref[...]
        covered = c > 0.0
        cf = covered.astype(jnp.float32)
        mx = jnp.max(jnp.where(covered, s, -1e30), axis=1)
        ws = jnp.sum(c * s, axis=1) * (1.0 / U)
        nc = jnp.sum(cf, axis=1)
        vv = v_ref[...]
        vcov = jnp.dot(cf, vv, preferred_element_type=jnp.float32)
        vtot = jnp.sum(vv, axis=0, keepdims=True)
        em = jnp.exp(ws - mx)
        denom = nc + em * (L - nc)
        z = (vcov + em[:, None] * (vtot - vcov)) / denom[:, None]
        y = x_ref[...] + z
        mu = jnp.mean(y, axis=1, keepdims=True)
        var = jnp.mean((y - mu) ** 2, axis=1, keepdims=True)
        o_ref[...] = (y - mu) * lax.rsqrt(var + 1e-5) * g_ref[...] + b_ref[...]

    return pl.pallas_call(
        body,
        grid=(L // tm,),
        in_specs=[
            pl.BlockSpec((tm, D), lambda i: (i, 0)),
            pl.BlockSpec((L, D), lambda i: (0, 0)),
            pl.BlockSpec((L, D), lambda i: (0, 0)),
            pl.BlockSpec((tm, L), lambda i: (i, 0)),
            pl.BlockSpec((tm, D), lambda i: (i, 0)),
            pl.BlockSpec((1, D), lambda i: (0, 0)),
            pl.BlockSpec((1, D), lambda i: (0, 0)),
        ],
        out_specs=pl.BlockSpec((tm, D), lambda i: (i, 0)),
        out_shape=jax.ShapeDtypeStruct((L, D), jnp.float32),
    )(q, k, v, cnt, x, g1, be1)


def _tc_ffn(x1, w1, b1, w2, b2, g2, be2, tm=256):
    inv_sqrt2 = 1.0 / math.sqrt(2.0)

    def body(x_ref, w1_ref, b1_ref, w2_ref, b2_ref, g_ref, be_ref, o_ref):
        xx = x_ref[...]
        h = (
            jnp.dot(xx, w1_ref[...], preferred_element_type=jnp.float32)
            + b1_ref[...]
        )
        h = 0.5 * h * (1.0 + lax.erf(h * inv_sqrt2))
        y = (
            jnp.dot(h, w2_ref[...], preferred_element_type=jnp.float32)
            + b2_ref[...]
            + xx
        )
        mu = jnp.mean(y, axis=1, keepdims=True)
        var = jnp.mean((y - mu) ** 2, axis=1, keepdims=True)
        o_ref[...] = (y - mu) * lax.rsqrt(var + 1e-5) * g_ref[...] + be_ref[...]

    return pl.pallas_call(
        body,
        grid=(L // tm,),
        in_specs=[
            pl.BlockSpec((tm, D), lambda i: (i, 0)),
            pl.BlockSpec((D, 4 * D), lambda i: (0, 0)),
            pl.BlockSpec((1, 4 * D), lambda i: (0, 0)),
            pl.BlockSpec((4 * D, D), lambda i: (0, 0)),
            pl.BlockSpec((1, D), lambda i: (0, 0)),
            pl.BlockSpec((1, D), lambda i: (0, 0)),
            pl.BlockSpec((1, D), lambda i: (0, 0)),
        ],
        out_specs=pl.BlockSpec((tm, D), lambda i: (i, 0)),
        out_shape=jax.ShapeDtypeStruct((L, D), jnp.float32),
    )(x1, w1, b1, w2, b2, g2, be2)


def kernel(x, idx, Wq, bq, Wk, bk, Wv, bv, W1, b1, W2, b2, g1, be1, g2, be2):
    x2d = x[0]
    cnt = _sc_histogram(idx.reshape(-1))
    wqkv = jnp.concatenate([Wq, Wk, Wv], axis=1)
    bqkv = jnp.concatenate([bq, bk, bv])[None, :]
    q, k, v = _tc_qkv(x2d, wqkv, bqkv)
    x1 = _tc_attn(q, k, v, cnt, x2d, g1[None], be1[None])
    x2 = _tc_ffn(x1, W1, b1[None], W2, b2[None], g2[None], be2[None])
    return x2[None]
